# baseline (speedup 1.0000x reference)
import functools

import jax
import jax.numpy as jnp
from jax import lax
from jax.experimental import pallas as pl
from jax.experimental.pallas import tpu as pltpu
from jax.experimental.pallas import tpu_sc as plsc

N_ENTITY = 1000000
DIM = 64
B = 16384
L2_COEF = 0.005

NC = 2
NS = 16
NW = NC * NS
BPW = B // NW
CHUNK = 128
NCHUNK = BPW // CHUNK
GROUPS = BPW // 16

_mesh = plsc.VectorSubcoreMesh(core_axis_name="c", subcore_axis_name="s")


@functools.partial(
    pl.kernel,
    out_type=(
        jax.ShapeDtypeStruct((NW, BPW), jnp.float32),
        jax.ShapeDtypeStruct((NW * 16,), jnp.float32),
    ),
    mesh=_mesh,
    compiler_params=pltpu.CompilerParams(
        needs_layout_passes=False, use_tc_tiling_on_sc=False
    ),
    scratch_types=(
        pltpu.VMEM((NCHUNK, CHUNK), jnp.int32),
        pltpu.VMEM((NCHUNK, CHUNK), jnp.int32),
        pltpu.VMEM((NCHUNK, CHUNK), jnp.int32),
        pltpu.VMEM((BPW, DIM), jnp.float32),
        pltpu.VMEM((BPW, DIM), jnp.float32),
        pltpu.VMEM((BPW, DIM), jnp.float32),
        pltpu.VMEM((BPW,), jnp.float32),
        pltpu.VMEM((256,), jnp.float32),
        pltpu.VMEM((16,), jnp.float32),
        pltpu.SemaphoreType.DMA,
    ),
)
def _sc_score(h_hbm, p_hbm, n_hbm, emb_hbm, diff_hbm, ss_hbm,
              h_idx, p_idx, n_idx, h_rows, p_rows, n_rows,
              diff_v, tp_v, ss_v, sem):
    wid = lax.axis_index("s") * NC + lax.axis_index("c")

    pltpu.sync_copy(h_hbm.at[wid], h_idx)
    pltpu.sync_copy(p_hbm.at[wid], p_idx)
    pltpu.sync_copy(n_hbm.at[wid], n_idx)

    copies = []
    for idx, rows in ((h_idx, h_rows), (p_idx, p_rows), (n_idx, n_rows)):
        for j in range(NCHUNK):
            copies.append(
                pltpu.async_copy(
                    emb_hbm.at[idx.at[j]],
                    rows.at[pl.ds(j * CHUNK, CHUNK)],
                    sem,
                )
            )
    for c in copies:
        c.wait()

    ss_v[...] = jnp.zeros((16,), jnp.float32)
    iota = lax.iota(jnp.int32, 16)

    def group_body(g, carry):
        base = g * 16
        sq = None
        for i in range(16):
            row = base + i
            d = None
            for c in range(DIM // 16):
                hh = h_rows[row, pl.ds(c * 16, 16)]
                pp = p_rows[row, pl.ds(c * 16, 16)]
                nn = n_rows[row, pl.ds(c * 16, 16)]
                t = hh * (nn - pp)
                d = t if d is None else d + t
                s = hh * hh + pp * pp + nn * nn
                sq = s if sq is None else sq + s
            plsc.store_scatter(tp_v, [iota * 16 + i], d)
        acc = None
        for k in range(16):
            v = tp_v[pl.ds(k * 16, 16)]
            acc = v if acc is None else acc + v
        diff_v[pl.ds(base, 16)] = acc
        ss_v[...] = ss_v[...] + sq
        return carry

    lax.fori_loop(0, GROUPS, group_body, 0)

    pltpu.sync_copy(diff_v, diff_hbm.at[wid])
    pltpu.sync_copy(ss_v, ss_hbm.at[pl.ds(wid * 16, 16)])


def _tc_body(diff_ref, ss_ref, out_ref):
    z = -diff_ref[...]
    sp = jnp.maximum(z, 0.0) + jnp.log1p(jnp.exp(-jnp.abs(z)))
    kg = jnp.sum(sp) * (1.0 / B)
    ss = jnp.sum(ss_ref[...])
    out_ref[0, 0] = kg + (L2_COEF * 0.5 / B) * ss


_tc_finish = pl.pallas_call(
    _tc_body,
    out_shape=jax.ShapeDtypeStruct((1, 1), jnp.float32),
    out_specs=pl.BlockSpec(memory_space=pltpu.SMEM),
)


def kernel(h, pos_t, neg_t, emb):
    h3 = h.astype(jnp.int32).reshape(NW, NCHUNK, CHUNK)
    p3 = pos_t.astype(jnp.int32).reshape(NW, NCHUNK, CHUNK)
    n3 = neg_t.astype(jnp.int32).reshape(NW, NCHUNK, CHUNK)
    diff, ss = _sc_score(h3, p3, n3, emb)
    loss = _tc_finish(diff.reshape(128, 128), ss.reshape(4, 128))
    return loss[0, 0]


# speedup vs baseline: 1.6848x; 1.6848x over previous
import functools

import jax
import jax.numpy as jnp
from jax import lax
from jax.experimental import pallas as pl
from jax.experimental.pallas import tpu as pltpu
from jax.experimental.pallas import tpu_sc as plsc

N_ENTITY = 1000000
DIM = 64
B = 16384
L2_COEF = 0.005

NC = 2
NS = 16
NW = NC * NS
BPW = B // NW
G = 16
NGROUP = BPW // G

_mesh = plsc.VectorSubcoreMesh(core_axis_name="c", subcore_axis_name="s")


@functools.partial(
    pl.kernel,
    out_type=jax.ShapeDtypeStruct((NW * 8, 128), jnp.float32),
    mesh=_mesh,
    compiler_params=pltpu.CompilerParams(needs_layout_passes=False),
    scratch_types=(
        pltpu.VMEM((8, 128), jnp.int32),
        pltpu.VMEM((8, 128), jnp.int32),
        pltpu.VMEM((8, 128), jnp.int32),
        pltpu.VMEM((2, G, DIM), jnp.float32),
        pltpu.VMEM((2, G, DIM), jnp.float32),
        pltpu.VMEM((2, G, DIM), jnp.float32),
        pltpu.VMEM((256,), jnp.float32),
        pltpu.VMEM((8, 128), jnp.float32),
        pltpu.SemaphoreType.DMA((2,)),
    ),
)
def _sc_score(h_hbm, p_hbm, n_hbm, emb_hbm, out_hbm,
              h_idx, p_idx, n_idx, h_buf, p_buf, n_buf,
              tp_v, out_v, sem):
    wid = lax.axis_index("s") * NC + lax.axis_index("c")

    pltpu.sync_copy(h_hbm.at[pl.ds(wid * 4, 4)], h_idx.at[pl.ds(0, 4)])
    pltpu.sync_copy(p_hbm.at[pl.ds(wid * 4, 4)], p_idx.at[pl.ds(0, 4)])
    pltpu.sync_copy(n_hbm.at[pl.ds(wid * 4, 4)], n_idx.at[pl.ds(0, 4)])

    streams = ((h_idx, h_buf), (p_idx, p_buf), (n_idx, n_buf))
    iota = lax.iota(jnp.int32, 16)

    def fire(g, p):
        row = lax.div(g, 8)
        col = lax.rem(g, 8) * 16
        for idx_v, buf in streams:
            tv = idx_v[row, pl.ds(col, 16)]
            for k in range(G):
                t = tv[k]
                pltpu.async_copy(
                    emb_hbm.at[pl.ds(t, 1)],
                    buf.at[p, pl.ds(k, 1)],
                    sem.at[p],
                )

    def drain(p):
        for _, buf in streams:
            for k in range(G):
                pltpu.make_async_copy(
                    emb_hbm.at[pl.ds(0, 1)],
                    buf.at[p, pl.ds(k, 1)],
                    sem.at[p],
                ).wait()

    fire(0, 0)

    def group_body(g, sq_carry):
        parity = lax.rem(g, 2)
        nxt = lax.rem(g + 1, 2)

        @pl.when(g < NGROUP - 1)
        def _():
            fire(g + 1, nxt)

        drain(parity)

        sq = sq_carry
        for k in range(G):
            d = None
            for c in range(DIM // 16):
                hh = h_buf[parity, k, pl.ds(c * 16, 16)]
                pp = p_buf[parity, k, pl.ds(c * 16, 16)]
                nn = n_buf[parity, k, pl.ds(c * 16, 16)]
                t = hh * (nn - pp)
                d = t if d is None else d + t
                sq = sq + (hh * hh + pp * pp + nn * nn)
            plsc.store_scatter(tp_v, [iota * 16 + k], d)
        acc = None
        for k in range(G):
            v = tp_v[pl.ds(k * 16, 16)]
            acc = v if acc is None else acc + v
        out_v[lax.div(g, 8), pl.ds(lax.rem(g, 8) * 16, 16)] = acc
        return sq

    sq_total = lax.fori_loop(
        0, NGROUP, group_body, jnp.zeros((16,), jnp.float32)
    )

    zeros16 = jnp.zeros((16,), jnp.float32)
    out_v[4, pl.ds(0, 16)] = sq_total
    for c in range(1, 8):
        out_v[4, pl.ds(c * 16, 16)] = zeros16
    for r in range(5, 8):
        for c in range(8):
            out_v[r, pl.ds(c * 16, 16)] = zeros16

    pltpu.sync_copy(out_v, out_hbm.at[pl.ds(wid * 8, 8)])


def _tc_body(x_ref, out_ref):
    x = x_ref[...]
    rows = lax.broadcasted_iota(jnp.int32, x.shape, 0)
    sub = rows % 8
    z = -x
    sp = jnp.maximum(z, 0.0) + jnp.log1p(jnp.exp(-jnp.abs(z)))
    kg = jnp.sum(jnp.where(sub < 4, sp, 0.0)) * (1.0 / B)
    ss = jnp.sum(jnp.where(sub == 4, x, 0.0))
    out_ref[0, 0] = kg + (L2_COEF * 0.5 / B) * ss


_tc_finish = pl.pallas_call(
    _tc_body,
    out_shape=jax.ShapeDtypeStruct((1, 1), jnp.float32),
    out_specs=pl.BlockSpec(memory_space=pltpu.SMEM),
)


def kernel(h, pos_t, neg_t, emb):
    h2 = h.astype(jnp.int32).reshape(128, 128)
    p2 = pos_t.astype(jnp.int32).reshape(128, 128)
    n2 = neg_t.astype(jnp.int32).reshape(128, 128)
    out = _sc_score(h2, p2, n2, emb)
    loss = _tc_finish(out)
    return loss[0, 0]
